# baseline (speedup 1.0000x reference)
import jax
import jax.numpy as jnp
from jax import lax
from jax.experimental import pallas as pl
from jax.experimental.pallas import tpu as pltpu
from jax.experimental.pallas import tpu_sc as plsc

_B = 16384
_EMBED = 64
_N_CONT = 13
_N_DISC = 26
_SLOTS = _N_CONT + _N_DISC
_VOCAB = 1000000

_NC = 2
_NS = 16
_NW = _NC * _NS
_BLK = 128
_BPW = _B // _BLK // _NW
_FROWS = _B * _SLOTS * _EMBED // 128


def _sc_body(ct_h, dt_h, ws_h, bs_h, ta_h, tb_h, out_h,
             dt_v, ct_v, ws_v, bs_v,
             ga0, ga1, gb0, gb1, tbuf0, tbuf1, cbuf0, cbuf1,
             sem_g0, sem_g1, sem_w0, sem_w1, sem_c0, sem_c1):
    c = lax.axis_index("c")
    s = lax.axis_index("s")
    wid = s * _NC + c
    b0 = wid * (_BPW * _BLK)

    for r in range(_N_DISC):
        pltpu.sync_copy(dt_h.at[r, pl.ds(b0, _BPW * _BLK)], dt_v.at[r])
    for r in range(_N_CONT):
        pltpu.sync_copy(ct_h.at[r, pl.ds(b0, _BPW * _BLK)], ct_v.at[r])
    pltpu.sync_copy(ws_h, ws_v)
    pltpu.sync_copy(bs_h, bs_v)

    io = lax.iota(jnp.int32, 16)
    gas = (ga0, ga1)
    gbs = (gb0, gb1)
    tbufs = (tbuf0, tbuf1)
    cbufs = (cbuf0, cbuf1)
    sems_g = (sem_g0, sem_g1)
    sems_w = (sem_w0, sem_w1)
    sems_c = (sem_c0, sem_c1)

    def fire_gather(sd, p, boff):
        idx = dt_v.at[sd, pl.ds(boff, _BLK)]
        pltpu.async_copy(ta_h.at[idx], gas[p], sems_g[p])
        pltpu.async_copy(tb_h.at[idx], gbs[p], sems_g[p])

    @pl.loop(0, _BPW)
    def _ublk(ublk):
        boff = pl.multiple_of(ublk * _BLK, _BLK)
        tcg = wid * _BPW + ublk
        not_first_blk = ublk > 0

        fire_gather(0, 0, boff)
        fire_gather(1, 1, boff)

        @pl.loop(0, _N_DISC, step=2)
        def _disc(sd0):
            for p in range(2):
                sd = sd0 + p
                tbuf = tbufs[p]

                pltpu.make_async_copy(
                    ta_h.at[pl.ds(0, _BLK)], gas[p], sems_g[p]).wait()
                pltpu.make_async_copy(
                    tb_h.at[pl.ds(0, _BLK)], gbs[p], sems_g[p]).wait()

                @pl.when(jnp.logical_or(not_first_blk, sd >= 2))
                def _():
                    pltpu.make_async_copy(
                        tbuf, out_h.at[pl.ds(0, _EMBED)], sems_w[p]).wait()

                for u in range(4):
                    gsrc = gas[p] if u < 2 else gbs[p]
                    csrc = 16 * (u % 2) + io
                    edst = 16 * u + io

                    @pl.loop(0, 8)
                    def _m(m):
                        mb = m * 16
                        for k in range(16):
                            rot = (io + k) & 15
                            bcol = mb + rot
                            v = plsc.load_gather(gsrc, [bcol, csrc])
                            plsc.store_scatter(tbuf, [edst, bcol], v)

                @pl.when(sd + 2 < _N_DISC)
                def _():
                    fire_gather(sd + 2, p, boff)

                base = ((sd + _N_CONT) * 8 * _BLK + tcg) * 8
                for tr in range(8):
                    pltpu.async_copy(
                        tbuf.at[pl.ds(tr * 8, 8)],
                        out_h.at[pl.ds(base + tr * _BLK * 8, 8)],
                        sems_w[p])

        def cont_compute_write(sc, p):
            cbuf = cbufs[p]
            cv = [ct_v[sc, pl.ds(boff + q * 16, 16)] for q in range(8)]

            @pl.loop(0, _EMBED, unroll=8)
            def _e(e):
                we = ws_v[e]
                be = bs_v[e]
                for q in range(8):
                    cbuf[e, pl.ds(q * 16, 16)] = we * cv[q] + be

            base = (sc * 8 * _BLK + tcg) * 8
            for tr in range(8):
                pltpu.async_copy(
                    cbuf.at[pl.ds(tr * 8, 8)],
                    out_h.at[pl.ds(base + tr * _BLK * 8, 8)],
                    sems_c[p])

        @pl.loop(0, _N_CONT - 1, step=2)
        def _cont(sc0):
            for p in range(2):
                sc = sc0 + p

                @pl.when(jnp.logical_or(not_first_blk, sc >= 2))
                def _():
                    pltpu.make_async_copy(
                        cbufs[p], out_h.at[pl.ds(0, _EMBED)],
                        sems_c[p]).wait()

                cont_compute_write(sc, p)

        pltpu.make_async_copy(
            cbufs[0], out_h.at[pl.ds(0, _EMBED)], sems_c[0]).wait()
        cont_compute_write(12, 0)

    for p in range(2):
        pltpu.make_async_copy(
            tbufs[p], out_h.at[pl.ds(0, _EMBED)], sems_w[p]).wait()
    pltpu.make_async_copy(
        cbufs[0], out_h.at[pl.ds(0, _EMBED)], sems_c[0]).wait()
    pltpu.make_async_copy(
        cbufs[1], out_h.at[pl.ds(0, _EMBED)], sems_c[1]).wait()


@jax.jit
def _mixed_input_sc(ct, dt, ws, bs, ta, tb):
    mesh = plsc.VectorSubcoreMesh(core_axis_name="c", subcore_axis_name="s")
    kfn = pl.kernel(
        _sc_body,
        out_type=jax.ShapeDtypeStruct((_FROWS, _BLK), jnp.float32),
        mesh=mesh,
        compiler_params=pltpu.CompilerParams(
            use_tc_tiling_on_sc=False, needs_layout_passes=False),
        scratch_types=[
            pltpu.VMEM((_N_DISC, _BPW * _BLK), jnp.int32),
            pltpu.VMEM((_N_CONT, _BPW * _BLK), jnp.float32),
            pltpu.VMEM((_EMBED, 16), jnp.float32),
            pltpu.VMEM((_EMBED, 16), jnp.float32),
            pltpu.VMEM((_BLK, _EMBED // 2), jnp.float32),
            pltpu.VMEM((_BLK, _EMBED // 2), jnp.float32),
            pltpu.VMEM((_BLK, _EMBED // 2), jnp.float32),
            pltpu.VMEM((_BLK, _EMBED // 2), jnp.float32),
            pltpu.VMEM((_EMBED, _BLK), jnp.float32),
            pltpu.VMEM((_EMBED, _BLK), jnp.float32),
            pltpu.VMEM((_EMBED, _BLK), jnp.float32),
            pltpu.VMEM((_EMBED, _BLK), jnp.float32),
            pltpu.SemaphoreType.DMA,
            pltpu.SemaphoreType.DMA,
            pltpu.SemaphoreType.DMA,
            pltpu.SemaphoreType.DMA,
            pltpu.SemaphoreType.DMA,
            pltpu.SemaphoreType.DMA,
        ],
    )
    return kfn(ct, dt, ws, bs, ta, tb)


def kernel(Continuous, Discrete, W_cont, b_cont, emb_table):
    ct = Continuous.T
    dt = Discrete.T
    wv = W_cont.reshape(_EMBED)
    ws = jnp.broadcast_to(wv[:, None], (_EMBED, 16))
    bs = jnp.broadcast_to(b_cont.reshape(_EMBED)[:, None], (_EMBED, 16))
    ta = emb_table[:, : _EMBED // 2]
    tb = emb_table[:, _EMBED // 2:]
    f = _mixed_input_sc(ct, dt, ws, bs, ta, tb)
    out = (f.reshape(_SLOTS, 8, _BLK, 8, _BLK)
           .transpose(2, 4, 0, 1, 3)
           .reshape(_B, _SLOTS, _EMBED))
    return out


# speedup vs baseline: 1.8479x; 1.8479x over previous
import jax
import jax.numpy as jnp
from jax import lax
from jax.experimental import pallas as pl
from jax.experimental.pallas import tpu as pltpu
from jax.experimental.pallas import tpu_sc as plsc

_B = 16384
_EMBED = 64
_N_CONT = 13
_N_DISC = 26
_SLOTS = _N_CONT + _N_DISC
_VOCAB = 1000000

_NC = 2
_NS = 16
_NW = _NC * _NS
_BLK = 128
_BPW = _B // _BLK // _NW
_FROWS = _B * _SLOTS * _EMBED // 128


def _sc_body(ct_h, dt_h, ws_h, bs_h, table_h, out_h,
             dt_v, ct_v, ws_v, bs_v, didx0, didx1,
             gbuf0, gbuf1, tbuf0, tbuf1, cbuf0, cbuf1,
             sem_g0, sem_g1, sem_w0, sem_w1, sem_c0, sem_c1):
    c = lax.axis_index("c")
    s = lax.axis_index("s")
    wid = s * _NC + c
    b0 = wid * (_BPW * _BLK)

    for r in range(_N_DISC):
        pltpu.sync_copy(dt_h.at[r, pl.ds(b0, _BPW * _BLK)], dt_v.at[r])
    for r in range(_N_CONT):
        pltpu.sync_copy(ct_h.at[r, pl.ds(b0, _BPW * _BLK)], ct_v.at[r])
    pltpu.sync_copy(ws_h, ws_v)
    pltpu.sync_copy(bs_h, bs_v)

    io = lax.iota(jnp.int32, 16)
    gbufs = (gbuf0, gbuf1)
    tbufs = (tbuf0, tbuf1)
    cbufs = (cbuf0, cbuf1)
    sems_g = (sem_g0, sem_g1)
    sems_w = (sem_w0, sem_w1)
    sems_c = (sem_c0, sem_c1)

    didxs = (didx0, didx1)

    def fire_gather(sd, p, boff):
        didx = didxs[p]
        io2 = io * 2
        for u in range(8):
            v2 = dt_v[sd, pl.ds(boff + u * 16, 16)] * 2
            plsc.store_scatter(didx, [io2 + 32 * u], v2)
            plsc.store_scatter(didx, [io2 + 32 * u + 1], v2 + 1)
        pltpu.async_copy(
            table_h.at[didx.at[pl.ds(0, _BLK)]],
            gbufs[p].at[pl.ds(0, _BLK)], sems_g[p])
        pltpu.async_copy(
            table_h.at[didx.at[pl.ds(_BLK, _BLK)]],
            gbufs[p].at[pl.ds(_BLK, _BLK)], sems_g[p])

    @pl.loop(0, _BPW)
    def _ublk(ublk):
        boff = pl.multiple_of(ublk * _BLK, _BLK)
        tcg = wid * _BPW + ublk
        not_first_blk = ublk > 0

        fire_gather(0, 0, boff)
        fire_gather(1, 1, boff)

        @pl.loop(0, _N_DISC, step=2)
        def _disc(sd0):
            for p in range(2):
                sd = sd0 + p
                gbuf = gbufs[p]
                tbuf = tbufs[p]

                pltpu.make_async_copy(
                    table_h.at[pl.ds(0, 2 * _BLK)], gbuf, sems_g[p]).wait()

                @pl.when(jnp.logical_or(not_first_blk, sd >= 2))
                def _():
                    pltpu.make_async_copy(
                        tbuf, out_h.at[pl.ds(0, _EMBED)], sems_w[p]).wait()

                for u in range(4):
                    h = u // 2
                    csrc = 16 * (u % 2) + io
                    edst = 16 * u + io

                    @pl.loop(0, 8)
                    def _m(m):
                        mb = m * 16
                        for k in range(16):
                            rot = (io + k) & 15
                            bcol = mb + rot
                            rowv = 2 * bcol + h
                            v = plsc.load_gather(gbuf, [rowv, csrc])
                            plsc.store_scatter(tbuf, [edst, bcol], v)

                @pl.when(sd + 2 < _N_DISC)
                def _():
                    fire_gather(sd + 2, p, boff)

                base = ((sd + _N_CONT) * 8 * _BLK + tcg) * 8
                for tr in range(8):
                    pltpu.async_copy(
                        tbuf.at[pl.ds(tr * 8, 8)],
                        out_h.at[pl.ds(base + tr * _BLK * 8, 8)],
                        sems_w[p])

        def cont_compute_write(sc, p):
            cbuf = cbufs[p]
            cv = [ct_v[sc, pl.ds(boff + q * 16, 16)] for q in range(8)]

            @pl.loop(0, _EMBED, unroll=8)
            def _e(e):
                we = ws_v[e]
                be = bs_v[e]
                for q in range(8):
                    cbuf[e, pl.ds(q * 16, 16)] = we * cv[q] + be

            base = (sc * 8 * _BLK + tcg) * 8
            for tr in range(8):
                pltpu.async_copy(
                    cbuf.at[pl.ds(tr * 8, 8)],
                    out_h.at[pl.ds(base + tr * _BLK * 8, 8)],
                    sems_c[p])

        @pl.loop(0, _N_CONT - 1, step=2)
        def _cont(sc0):
            for p in range(2):
                sc = sc0 + p

                @pl.when(jnp.logical_or(not_first_blk, sc >= 2))
                def _():
                    pltpu.make_async_copy(
                        cbufs[p], out_h.at[pl.ds(0, _EMBED)],
                        sems_c[p]).wait()

                cont_compute_write(sc, p)

        pltpu.make_async_copy(
            cbufs[0], out_h.at[pl.ds(0, _EMBED)], sems_c[0]).wait()
        cont_compute_write(12, 0)

    for p in range(2):
        pltpu.make_async_copy(
            tbufs[p], out_h.at[pl.ds(0, _EMBED)], sems_w[p]).wait()
    pltpu.make_async_copy(
        cbufs[0], out_h.at[pl.ds(0, _EMBED)], sems_c[0]).wait()
    pltpu.make_async_copy(
        cbufs[1], out_h.at[pl.ds(0, _EMBED)], sems_c[1]).wait()


@jax.jit
def _mixed_input_sc(ct, dt, ws, bs, table):
    mesh = plsc.VectorSubcoreMesh(core_axis_name="c", subcore_axis_name="s")
    kfn = pl.kernel(
        _sc_body,
        out_type=jax.ShapeDtypeStruct((_FROWS, _BLK), jnp.float32),
        mesh=mesh,
        compiler_params=pltpu.CompilerParams(
            use_tc_tiling_on_sc=False, needs_layout_passes=False),
        scratch_types=[
            pltpu.VMEM((_N_DISC, _BPW * _BLK), jnp.int32),
            pltpu.VMEM((_N_CONT, _BPW * _BLK), jnp.float32),
            pltpu.VMEM((_EMBED, 16), jnp.float32),
            pltpu.VMEM((_EMBED, 16), jnp.float32),
            pltpu.VMEM((2 * _BLK,), jnp.int32),
            pltpu.VMEM((2 * _BLK,), jnp.int32),
            pltpu.VMEM((2 * _BLK, _EMBED // 2), jnp.float32),
            pltpu.VMEM((2 * _BLK, _EMBED // 2), jnp.float32),
            pltpu.VMEM((_EMBED, _BLK), jnp.float32),
            pltpu.VMEM((_EMBED, _BLK), jnp.float32),
            pltpu.VMEM((_EMBED, _BLK), jnp.float32),
            pltpu.VMEM((_EMBED, _BLK), jnp.float32),
            pltpu.SemaphoreType.DMA,
            pltpu.SemaphoreType.DMA,
            pltpu.SemaphoreType.DMA,
            pltpu.SemaphoreType.DMA,
            pltpu.SemaphoreType.DMA,
            pltpu.SemaphoreType.DMA,
        ],
    )
    return kfn(ct, dt, ws, bs, table)


def kernel(Continuous, Discrete, W_cont, b_cont, emb_table):
    ct = Continuous.T
    dt = Discrete.T
    wv = W_cont.reshape(_EMBED)
    ws = jnp.broadcast_to(wv[:, None], (_EMBED, 16))
    bs = jnp.broadcast_to(b_cont.reshape(_EMBED)[:, None], (_EMBED, 16))
    tlin = emb_table.reshape(2 * _VOCAB, _EMBED // 2)
    f = _mixed_input_sc(ct, dt, ws, bs, tlin)
    out = (f.reshape(_SLOTS, 8, _BLK, 8, _BLK)
           .transpose(2, 4, 0, 1, 3)
           .reshape(_B, _SLOTS, _EMBED))
    return out
